# baseline (speedup 1.0000x reference)
import functools

import jax
import jax.numpy as jnp
from jax import lax
from jax.experimental import pallas as pl
from jax.experimental.pallas import tpu as pltpu
from jax.experimental.pallas import tpu_sc as plsc

T, D, H, O, E = 2048, 768, 2048, 768, 8
TOPK = 2
BM = 512
NB = (TOPK * T) // BM + E
P = NB * BM
HC = 256

NC, NS = 2, 16
NW = NC * NS
XTPW = T // NW
CPW = T // NW
CCH = 16
LN = 16


def _router_body(x_ref, wg_ref, bg_ref, p0_ref, p1_ref, w0_ref, w1_ref,
                 be_ref, nbu_ref):
    x = x_ref[...]
    logits = jnp.dot(x, wg_ref[...], preferred_element_type=jnp.float32)
    logits = logits + bg_ref[...]
    m = jnp.max(logits, axis=1, keepdims=True)
    ex = jnp.exp(logits - m)
    p = ex / jnp.sum(ex, axis=1, keepdims=True)
    it = lax.broadcasted_iota(jnp.int32, (T, E), 1)
    m1 = jnp.max(p, axis=1, keepdims=True)
    a1 = jnp.min(jnp.where(p == m1, it, E), axis=1)
    pm = jnp.where(it == a1[:, None], -jnp.inf, p)
    m2 = jnp.max(pm, axis=1, keepdims=True)
    a2 = jnp.min(jnp.where(pm == m2, it, E), axis=1)
    denom = jnp.maximum(m1 + m2, 1e-12)
    w0_ref[...] = m1 / denom
    w1_ref[...] = m2 / denom

    c0 = (it == a1[:, None]).astype(jnp.int32)
    c1 = (it == a2[:, None]).astype(jnp.int32)
    mm = c0 + c1
    s = mm
    k = 1
    while k < T:
        s = s + jnp.concatenate(
            [jnp.zeros((k, E), jnp.int32), s[:T - k, :]], axis=0)
        k *= 2
    sex = s - mm
    counts = s[T - 1:T, :]
    nblk = (counts + (BM - 1)) // BM
    pi = nblk
    k = 1
    while k < E:
        pi = pi + jnp.concatenate(
            [jnp.zeros((1, k), jnp.int32), pi[:, :E - k]], axis=1)
        k *= 2
    po = BM * (pi - nblk)
    base = po + sex
    p0_ref[...] = jnp.sum(c0 * base, axis=1, keepdims=True)
    p1_ref[...] = jnp.sum(c1 * base, axis=1, keepdims=True)
    starts = BM * lax.broadcasted_iota(jnp.int32, (NB, E), 0)
    be = jnp.sum((po <= starts).astype(jnp.int32), axis=1, keepdims=True) - 1
    be_ref[...] = be
    nbu_ref[...] = jnp.sum(nblk, axis=1, keepdims=True)


def _router(x2, Wg, bg):
    return pl.pallas_call(
        _router_body,
        out_shape=(jax.ShapeDtypeStruct((T, 1), jnp.int32),
                   jax.ShapeDtypeStruct((T, 1), jnp.int32),
                   jax.ShapeDtypeStruct((T, 1), jnp.float32),
                   jax.ShapeDtypeStruct((T, 1), jnp.float32),
                   jax.ShapeDtypeStruct((NB, 1), jnp.int32),
                   jax.ShapeDtypeStruct((1, 1), jnp.int32)),
    )(x2, Wg, bg.reshape(1, E))


def _sc_dispatch(x2, pos0, pos1, w0, w1):
    mesh = plsc.VectorSubcoreMesh(core_axis_name="c", subcore_axis_name="s")

    @functools.partial(
        pl.kernel, mesh=mesh,
        out_type=(jax.ShapeDtypeStruct((P, D), jnp.float32),
                  jax.ShapeDtypeStruct((P,), jnp.float32)),
        scratch_types=[
            pltpu.VMEM((XTPW,), jnp.int32),
            pltpu.VMEM((XTPW,), jnp.int32),
            pltpu.VMEM((XTPW,), jnp.float32),
            pltpu.VMEM((XTPW,), jnp.float32),
            pltpu.VMEM((XTPW, D), jnp.float32),
            pltpu.SemaphoreType.DMA,
            pltpu.SemaphoreType.DMA,
        ],
    )
    def k(x_hbm, p0_hbm, p1_hbm, w0_hbm, w1_hbm, xs_hbm, wp_hbm,
          idx0_v, idx1_v, w0_v, w1_v, rows_v, s0, s1):
        wid = lax.axis_index("s") * NC + lax.axis_index("c")
        tb = wid * XTPW
        cp = pltpu.async_copy(x_hbm.at[pl.ds(tb, XTPW)], rows_v, s0)
        pltpu.sync_copy(p0_hbm.at[pl.ds(tb, XTPW)], idx0_v)
        pltpu.sync_copy(p1_hbm.at[pl.ds(tb, XTPW)], idx1_v)
        pltpu.sync_copy(w0_hbm.at[pl.ds(tb, XTPW)], w0_v)
        pltpu.sync_copy(w1_hbm.at[pl.ds(tb, XTPW)], w1_v)
        wa = pltpu.async_copy(w0_v, wp_hbm.at[idx0_v], s1)
        wb = pltpu.async_copy(w1_v, wp_hbm.at[idx1_v], s1)
        cp.wait()
        a = pltpu.async_copy(rows_v, xs_hbm.at[idx0_v], s0)
        b = pltpu.async_copy(rows_v, xs_hbm.at[idx1_v], s0)
        wa.wait()
        wb.wait()
        a.wait()
        b.wait()

    return k(x2, pos0, pos1, w0, w1)


def _gmm_body(be_ref, nu_ref, xs_ref, wp_ref, W1_ref, b1_ref, W2_ref, b2_ref,
              out_ref, acc_ref):
    b = pl.program_id(0)

    @pl.when(b < nu_ref[0])
    def _():
        xb = xs_ref[...].astype(jnp.bfloat16)
        acc_ref[...] = jnp.zeros((BM, O), jnp.float32)
        for k0 in range(H // HC):
            w1c = W1_ref[0, :, k0 * HC:(k0 + 1) * HC].astype(jnp.bfloat16)
            b1c = b1_ref[0, 0, k0 * HC:(k0 + 1) * HC]
            hc = jnp.maximum(
                jnp.dot(xb, w1c, preferred_element_type=jnp.float32)
                + b1c[None, :], 0.0).astype(jnp.bfloat16)
            acc_ref[...] += jnp.dot(
                hc, W2_ref[0, k0 * HC:(k0 + 1) * HC, :].astype(jnp.bfloat16),
                preferred_element_type=jnp.float32)
        out_ref[...] = ((acc_ref[...] + b2_ref[0, 0, :][None, :])
                        * wp_ref[...])


def _gmm(xs, wp, W1, b1, W2, b2, blk_expert, nb_used):
    gs = pltpu.PrefetchScalarGridSpec(
        num_scalar_prefetch=2,
        grid=(NB,),
        in_specs=[
            pl.BlockSpec((BM, D),
                         lambda b, be, nu: (jnp.where(b < nu[0], b, 0), 0)),
            pl.BlockSpec((BM, 1),
                         lambda b, be, nu: (jnp.where(b < nu[0], b, 0), 0)),
            pl.BlockSpec((1, D, H), lambda b, be, nu: (be[b], 0, 0)),
            pl.BlockSpec((1, 1, H), lambda b, be, nu: (be[b], 0, 0)),
            pl.BlockSpec((1, H, O), lambda b, be, nu: (be[b], 0, 0)),
            pl.BlockSpec((1, 1, O), lambda b, be, nu: (be[b], 0, 0)),
        ],
        out_specs=pl.BlockSpec(
            (BM, O), lambda b, be, nu: (jnp.where(b < nu[0], b, NB), 0)),
        scratch_shapes=[pltpu.VMEM((BM, O), jnp.float32)],
    )
    return pl.pallas_call(
        _gmm_body, grid_spec=gs,
        out_shape=jax.ShapeDtypeStruct(((NB + 1) * BM, O), jnp.float32),
    )(blk_expert, nb_used, xs, wp.reshape(P, 1), W1, b1.reshape(E, 1, H), W2,
      b2.reshape(E, 1, O))


def _sc_combine(ys, pos0, pos1):
    mesh = plsc.VectorSubcoreMesh(core_axis_name="c", subcore_axis_name="s")

    @functools.partial(
        pl.kernel, mesh=mesh,
        out_type=jax.ShapeDtypeStruct((T, O), jnp.float32),
        scratch_types=[
            pltpu.VMEM((CPW,), jnp.int32),
            pltpu.VMEM((CPW,), jnp.int32),
            pltpu.VMEM((CCH, O), jnp.float32),
            pltpu.VMEM((CCH, O), jnp.float32),
            pltpu.VMEM((CCH, O), jnp.float32),
            pltpu.VMEM((CCH, O), jnp.float32),
            pltpu.VMEM((CCH, O), jnp.float32),
            pltpu.VMEM((CCH, O), jnp.float32),
            pltpu.SemaphoreType.DMA,
            pltpu.SemaphoreType.DMA,
        ],
    )
    def k(ys_hbm, p0_hbm, p1_hbm, out_hbm,
          idx0_v, idx1_v, ra0, ra1, rb0, rb1, oa, ob, gsem, wsem):
        wid = lax.axis_index("s") * NC + lax.axis_index("c")
        tb = wid * CPW
        pltpu.sync_copy(p0_hbm.at[pl.ds(tb, CPW)], idx0_v)
        pltpu.sync_copy(p1_hbm.at[pl.ds(tb, CPW)], idx1_v)
        rabufs = (ra0, ra1)
        rbbufs = (rb0, rb1)
        obufs = (oa, ob)
        nch = CPW // CCH

        def start_gather(c):
            sl = pl.ds(c * CCH, CCH)
            return (pltpu.async_copy(ys_hbm.at[idx0_v.at[sl]],
                                     rabufs[c % 2], gsem),
                    pltpu.async_copy(ys_hbm.at[idx1_v.at[sl]],
                                     rbbufs[c % 2], gsem))

        gathers = [None] * nch
        gathers[0] = start_gather(0)
        writes = []
        for c in range(nch):
            ga, gb = gathers[c]
            ga.wait()
            gb.wait()
            if c + 1 < nch:
                gathers[c + 1] = start_gather(c + 1)
            ra = rabufs[c % 2]
            rb = rbbufs[c % 2]
            out_v = obufs[c % 2]
            if c >= 2:
                writes[c - 2].wait()

            def tok_body(i, carry):
                for j in range(O // LN):
                    sl = pl.ds(j * LN, LN)
                    out_v[i, sl] = ra[i, sl] + rb[i, sl]
                return carry

            lax.fori_loop(0, CCH, tok_body, 0)
            writes.append(pltpu.async_copy(
                out_v, out_hbm.at[pl.ds(tb + c * CCH, CCH)], wsem))
        for wcp in writes[-2:]:
            wcp.wait()

    return k(ys, pos0, pos1)


def kernel(x, Wg, bg, W1, b1, W2, b2, num_experts_per_tok):
    del num_experts_per_tok
    x2 = x.reshape(T, D)
    p0, p1, w0, w1, be, nbu = _router(x2, Wg, bg)
    pos0 = p0.reshape(T)
    pos1 = p1.reshape(T)
    xs, wp = _sc_dispatch(x2, pos0, pos1, w0.reshape(T), w1.reshape(T))
    ys = _gmm(xs, wp, W1, b1, W2, b2, be.reshape(NB), nbu.reshape(1))
    out = _sc_combine(ys, pos0, pos1)
    return out.reshape(1, T, O)


# speedup vs baseline: 1.2913x; 1.2913x over previous
import functools

import jax
import jax.numpy as jnp
from jax import lax
from jax.experimental import pallas as pl
from jax.experimental.pallas import tpu as pltpu
from jax.experimental.pallas import tpu_sc as plsc

T, D, H, O, E = 2048, 768, 2048, 768, 8
TOPK = 2
BM = 512
NB = (TOPK * T) // BM + E
P = NB * BM
HC = 256

NC, NS = 2, 16
NW = NC * NS
XTPW = T // NW
CPW = T // NW
CCH = 16
LN = 16


def _router_body(x_ref, wg_ref, bg_ref, p0_ref, p1_ref, w0_ref, w1_ref,
                 be_ref, nbu_ref, br_ref):
    x = x_ref[...]
    logits = jnp.dot(x, wg_ref[...], preferred_element_type=jnp.float32)
    logits = logits + bg_ref[...]
    m = jnp.max(logits, axis=1, keepdims=True)
    ex = jnp.exp(logits - m)
    p = ex / jnp.sum(ex, axis=1, keepdims=True)
    it = lax.broadcasted_iota(jnp.int32, (T, E), 1)
    m1 = jnp.max(p, axis=1, keepdims=True)
    a1 = jnp.min(jnp.where(p == m1, it, E), axis=1)
    pm = jnp.where(it == a1[:, None], -jnp.inf, p)
    m2 = jnp.max(pm, axis=1, keepdims=True)
    a2 = jnp.min(jnp.where(pm == m2, it, E), axis=1)
    denom = jnp.maximum(m1 + m2, 1e-12)
    w0_ref[...] = jnp.broadcast_to(m1 / denom, (T, LN))
    w1_ref[...] = jnp.broadcast_to(m2 / denom, (T, LN))

    c0 = (it == a1[:, None]).astype(jnp.int32)
    c1 = (it == a2[:, None]).astype(jnp.int32)
    mm = c0 + c1
    s = mm
    k = 1
    while k < T:
        s = s + jnp.concatenate(
            [jnp.zeros((k, E), jnp.int32), s[:T - k, :]], axis=0)
        k *= 2
    sex = s - mm
    counts = s[T - 1:T, :]
    nblk = (counts + (BM - 1)) // BM
    pi = nblk
    k = 1
    while k < E:
        pi = pi + jnp.concatenate(
            [jnp.zeros((1, k), jnp.int32), pi[:, :E - k]], axis=1)
        k *= 2
    po = BM * (pi - nblk)
    base = po + sex
    p0_ref[...] = jnp.sum(c0 * base, axis=1, keepdims=True)
    p1_ref[...] = jnp.sum(c1 * base, axis=1, keepdims=True)
    starts = BM * lax.broadcasted_iota(jnp.int32, (NB, E), 0)
    bemask = (po <= starts).astype(jnp.int32)
    be = jnp.sum(bemask, axis=1, keepdims=True) - 1
    be_ref[...] = be
    nbu_ref[...] = jnp.sum(nblk, axis=1, keepdims=True)
    eix = lax.broadcasted_iota(jnp.int32, (NB, E), 1)
    own = (eix == be).astype(jnp.int32)
    ce = jnp.sum(own * counts, axis=1, keepdims=True)
    pb = jnp.sum(own * po, axis=1, keepdims=True)
    br_ref[...] = jnp.clip(ce - (starts[:, :1] - pb), 0, BM)


def _router(x2, Wg, bg):
    return pl.pallas_call(
        _router_body,
        out_shape=(jax.ShapeDtypeStruct((T, 1), jnp.int32),
                   jax.ShapeDtypeStruct((T, 1), jnp.int32),
                   jax.ShapeDtypeStruct((T, LN), jnp.float32),
                   jax.ShapeDtypeStruct((T, LN), jnp.float32),
                   jax.ShapeDtypeStruct((NB, 1), jnp.int32),
                   jax.ShapeDtypeStruct((1, 1), jnp.int32),
                   jax.ShapeDtypeStruct((NB, 1), jnp.int32)),
    )(x2, Wg, bg.reshape(1, E))


def _sc_dispatch(x2, pos0, pos1):
    mesh = plsc.VectorSubcoreMesh(core_axis_name="c", subcore_axis_name="s")

    @functools.partial(
        pl.kernel, mesh=mesh,
        out_type=jax.ShapeDtypeStruct((P, D), jnp.float32),
        scratch_types=[
            pltpu.VMEM((XTPW,), jnp.int32),
            pltpu.VMEM((XTPW,), jnp.int32),
            pltpu.VMEM((XTPW, D), jnp.float32),
            pltpu.SemaphoreType.DMA,
            pltpu.SemaphoreType.DMA,
        ],
    )
    def k(x_hbm, p0_hbm, p1_hbm, xs_hbm, idx0_v, idx1_v, rows_v, s0, s1):
        wid = lax.axis_index("s") * NC + lax.axis_index("c")
        tb = wid * XTPW
        cp = pltpu.async_copy(x_hbm.at[pl.ds(tb, XTPW)], rows_v, s0)
        pltpu.sync_copy(p0_hbm.at[pl.ds(tb, XTPW)], idx0_v)
        pltpu.sync_copy(p1_hbm.at[pl.ds(tb, XTPW)], idx1_v)
        cp.wait()
        a = pltpu.async_copy(rows_v, xs_hbm.at[idx0_v], s0)
        b = pltpu.async_copy(rows_v, xs_hbm.at[idx1_v], s1)
        a.wait()
        b.wait()

    return k(x2, pos0, pos1)


HB = BM // 2


def _gmm_body(be_ref, nu_ref, br_ref, xs_ref, W1_ref, b1_ref, W2_ref, b2_ref,
              out_ref, acc_ref):
    b = pl.program_id(0)

    def half(r0):
        xb = xs_ref[pl.ds(r0, HB), :].astype(jnp.bfloat16)
        acc_ref[...] = jnp.zeros((HB, O), jnp.float32)
        for k0 in range(H // HC):
            w1c = W1_ref[0, :, k0 * HC:(k0 + 1) * HC].astype(jnp.bfloat16)
            b1c = b1_ref[0, 0, k0 * HC:(k0 + 1) * HC]
            hc = jnp.maximum(
                jnp.dot(xb, w1c, preferred_element_type=jnp.float32)
                + b1c[None, :], 0.0).astype(jnp.bfloat16)
            acc_ref[...] += jnp.dot(
                hc, W2_ref[0, k0 * HC:(k0 + 1) * HC, :].astype(jnp.bfloat16),
                preferred_element_type=jnp.float32)
        out_ref[pl.ds(r0, HB), :] = acc_ref[...] + b2_ref[0, 0, :][None, :]

    @pl.when(b < nu_ref[0])
    def _():
        half(0)

    @pl.when((b < nu_ref[0]) & (br_ref[b] > HB))
    def _():
        half(HB)


def _gmm(xs, W1, b1, W2, b2, blk_expert, nb_used, blk_rows):
    gs = pltpu.PrefetchScalarGridSpec(
        num_scalar_prefetch=3,
        grid=(NB,),
        in_specs=[
            pl.BlockSpec((BM, D),
                         lambda b, be, nu, br: (jnp.where(b < nu[0], b, 0),
                                                0)),
            pl.BlockSpec((1, D, H), lambda b, be, nu, br: (be[b], 0, 0)),
            pl.BlockSpec((1, 1, H), lambda b, be, nu, br: (be[b], 0, 0)),
            pl.BlockSpec((1, H, O), lambda b, be, nu, br: (be[b], 0, 0)),
            pl.BlockSpec((1, 1, O), lambda b, be, nu, br: (be[b], 0, 0)),
        ],
        out_specs=pl.BlockSpec(
            (BM, O), lambda b, be, nu, br: (jnp.where(b < nu[0], b, NB), 0)),
        scratch_shapes=[pltpu.VMEM((HB, O), jnp.float32)],
    )
    return pl.pallas_call(
        _gmm_body, grid_spec=gs,
        out_shape=jax.ShapeDtypeStruct(((NB + 1) * BM, O), jnp.float32),
    )(blk_expert, nb_used, blk_rows, xs, W1, b1.reshape(E, 1, H), W2,
      b2.reshape(E, 1, O))


def _sc_combine(ys, pos0, pos1, w0b, w1b):
    mesh = plsc.VectorSubcoreMesh(core_axis_name="c", subcore_axis_name="s")

    @functools.partial(
        pl.kernel, mesh=mesh,
        out_type=jax.ShapeDtypeStruct((T, O), jnp.float32),
        scratch_types=[
            pltpu.VMEM((CPW,), jnp.int32),
            pltpu.VMEM((CPW,), jnp.int32),
            pltpu.VMEM((CPW, LN), jnp.float32),
            pltpu.VMEM((CPW, LN), jnp.float32),
            pltpu.VMEM((CCH, O), jnp.float32),
            pltpu.VMEM((CCH, O), jnp.float32),
            pltpu.VMEM((CCH, O), jnp.float32),
            pltpu.VMEM((CCH, O), jnp.float32),
            pltpu.VMEM((CCH, O), jnp.float32),
            pltpu.VMEM((CCH, O), jnp.float32),
            pltpu.SemaphoreType.DMA,
            pltpu.SemaphoreType.DMA,
        ],
    )
    def k(ys_hbm, p0_hbm, p1_hbm, w0_hbm, w1_hbm, out_hbm,
          idx0_v, idx1_v, w0_v, w1_v, ra0, ra1, rb0, rb1, oa, ob,
          gsem, wsem):
        wid = lax.axis_index("s") * NC + lax.axis_index("c")
        tb = wid * CPW
        pltpu.sync_copy(p0_hbm.at[pl.ds(tb, CPW)], idx0_v)
        pltpu.sync_copy(p1_hbm.at[pl.ds(tb, CPW)], idx1_v)
        pltpu.sync_copy(w0_hbm.at[pl.ds(tb, CPW)], w0_v)
        pltpu.sync_copy(w1_hbm.at[pl.ds(tb, CPW)], w1_v)
        rabufs = (ra0, ra1)
        rbbufs = (rb0, rb1)
        obufs = (oa, ob)
        nch = CPW // CCH

        def start_gather(c):
            sl = pl.ds(c * CCH, CCH)
            return (pltpu.async_copy(ys_hbm.at[idx0_v.at[sl]],
                                     rabufs[c % 2], gsem),
                    pltpu.async_copy(ys_hbm.at[idx1_v.at[sl]],
                                     rbbufs[c % 2], gsem))

        gathers = [None] * nch
        gathers[0] = start_gather(0)
        writes = []
        for c in range(nch):
            ga, gb = gathers[c]
            ga.wait()
            gb.wait()
            if c + 1 < nch:
                gathers[c + 1] = start_gather(c + 1)
            ra = rabufs[c % 2]
            rb = rbbufs[c % 2]
            out_v = obufs[c % 2]
            cbase = c * CCH
            if c >= 2:
                writes[c - 2].wait()

            def tok_body(i, carry):
                w0 = w0_v[cbase + i, :]
                w1 = w1_v[cbase + i, :]
                for j in range(O // LN):
                    sl = pl.ds(j * LN, LN)
                    out_v[i, sl] = w0 * ra[i, sl] + w1 * rb[i, sl]
                return carry

            lax.fori_loop(0, CCH, tok_body, 0)
            writes.append(pltpu.async_copy(
                out_v, out_hbm.at[pl.ds(tb + c * CCH, CCH)], wsem))
        for wcp in writes[-2:]:
            wcp.wait()

    return k(ys, pos0, pos1, w0b, w1b)


def kernel(x, Wg, bg, W1, b1, W2, b2, num_experts_per_tok):
    del num_experts_per_tok
    x2 = x.reshape(T, D)
    p0, p1, w0b, w1b, be, nbu, br = _router(x2, Wg, bg)
    pos0 = p0.reshape(T)
    pos1 = p1.reshape(T)
    xs = _sc_dispatch(x2, pos0, pos1)
    ys = _gmm(xs, W1, b1, W2, b2, be.reshape(NB), nbu.reshape(1),
              br.reshape(NB))
    out = _sc_combine(ys, pos0, pos1, w0b, w1b)
    return out.reshape(1, T, O)


# speedup vs baseline: 1.3004x; 1.0070x over previous
import functools

import jax
import jax.numpy as jnp
from jax import lax
from jax.experimental import pallas as pl
from jax.experimental.pallas import tpu as pltpu
from jax.experimental.pallas import tpu_sc as plsc

T, D, H, O, E = 2048, 768, 2048, 768, 8
TOPK = 2
BM = 512
NB = (TOPK * T) // BM + E
P = NB * BM
HC = 256

NC, NS = 2, 16
NW = NC * NS
XTPW = T // NW
CPW = T // NW
CCH = 16
LN = 16


def _router_body(x_ref, wg_ref, bg_ref, p0_ref, p1_ref, w0_ref, w1_ref,
                 be_ref, nbu_ref):
    x = x_ref[...]
    logits = jnp.dot(x, wg_ref[...], preferred_element_type=jnp.float32)
    logits = logits + bg_ref[...]
    m = jnp.max(logits, axis=1, keepdims=True)
    ex = jnp.exp(logits - m)
    p = ex / jnp.sum(ex, axis=1, keepdims=True)
    it = lax.broadcasted_iota(jnp.int32, (T, E), 1)
    m1 = jnp.max(p, axis=1, keepdims=True)
    a1 = jnp.min(jnp.where(p == m1, it, E), axis=1)
    pm = jnp.where(it == a1[:, None], -jnp.inf, p)
    m2 = jnp.max(pm, axis=1, keepdims=True)
    a2 = jnp.min(jnp.where(pm == m2, it, E), axis=1)
    denom = jnp.maximum(m1 + m2, 1e-12)
    w0_ref[...] = jnp.broadcast_to(m1 / denom, (T, LN))
    w1_ref[...] = jnp.broadcast_to(m2 / denom, (T, LN))

    c0 = (it == a1[:, None]).astype(jnp.int32)
    c1 = (it == a2[:, None]).astype(jnp.int32)
    mm = c0 + c1
    s = mm
    k = 1
    while k < T:
        s = s + jnp.concatenate(
            [jnp.zeros((k, E), jnp.int32), s[:T - k, :]], axis=0)
        k *= 2
    sex = s - mm
    counts = s[T - 1:T, :]
    nblk = (counts + (BM - 1)) // BM
    pi = nblk
    k = 1
    while k < E:
        pi = pi + jnp.concatenate(
            [jnp.zeros((1, k), jnp.int32), pi[:, :E - k]], axis=1)
        k *= 2
    po = BM * (pi - nblk)
    base = po + sex
    p0_ref[...] = jnp.sum(c0 * base, axis=1, keepdims=True)
    p1_ref[...] = jnp.sum(c1 * base, axis=1, keepdims=True)
    starts = BM * lax.broadcasted_iota(jnp.int32, (NB, E), 0)
    be = jnp.sum((po <= starts).astype(jnp.int32), axis=1, keepdims=True) - 1
    be_ref[...] = be
    nbu_ref[...] = jnp.sum(nblk, axis=1, keepdims=True)


def _router(x2, Wg, bg):
    return pl.pallas_call(
        _router_body,
        out_shape=(jax.ShapeDtypeStruct((T, 1), jnp.int32),
                   jax.ShapeDtypeStruct((T, 1), jnp.int32),
                   jax.ShapeDtypeStruct((T, LN), jnp.float32),
                   jax.ShapeDtypeStruct((T, LN), jnp.float32),
                   jax.ShapeDtypeStruct((NB, 1), jnp.int32),
                   jax.ShapeDtypeStruct((1, 1), jnp.int32)),
    )(x2, Wg, bg.reshape(1, E))


def _sc_dispatch(x2, pos0, pos1):
    mesh = plsc.VectorSubcoreMesh(core_axis_name="c", subcore_axis_name="s")

    @functools.partial(
        pl.kernel, mesh=mesh,
        out_type=jax.ShapeDtypeStruct((P, D), jnp.float32),
        scratch_types=[
            pltpu.VMEM((XTPW,), jnp.int32),
            pltpu.VMEM((XTPW,), jnp.int32),
            pltpu.VMEM((XTPW, D), jnp.float32),
            pltpu.SemaphoreType.DMA,
            pltpu.SemaphoreType.DMA,
        ],
    )
    def k(x_hbm, p0_hbm, p1_hbm, xs_hbm, idx0_v, idx1_v, rows_v, s0, s1):
        wid = lax.axis_index("s") * NC + lax.axis_index("c")
        tb = wid * XTPW
        cp = pltpu.async_copy(x_hbm.at[pl.ds(tb, XTPW)], rows_v, s0)
        pltpu.sync_copy(p0_hbm.at[pl.ds(tb, XTPW)], idx0_v)
        pltpu.sync_copy(p1_hbm.at[pl.ds(tb, XTPW)], idx1_v)
        cp.wait()
        a = pltpu.async_copy(rows_v, xs_hbm.at[idx0_v], s0)
        b = pltpu.async_copy(rows_v, xs_hbm.at[idx1_v], s1)
        a.wait()
        b.wait()

    return k(x2, pos0, pos1)


def _gmm_body(be_ref, nu_ref, xs_ref, W1_ref, b1_ref, W2_ref, b2_ref,
              out_ref, acc_ref):
    b = pl.program_id(0)

    @pl.when(b < nu_ref[0])
    def _():
        xb = xs_ref[...].astype(jnp.bfloat16)
        acc_ref[...] = jnp.zeros((BM, O), jnp.float32)
        for k0 in range(H // HC):
            w1c = W1_ref[0, :, k0 * HC:(k0 + 1) * HC].astype(jnp.bfloat16)
            b1c = b1_ref[0, 0, k0 * HC:(k0 + 1) * HC]
            hc = jnp.maximum(
                jnp.dot(xb, w1c, preferred_element_type=jnp.float32)
                + b1c[None, :], 0.0).astype(jnp.bfloat16)
            acc_ref[...] += jnp.dot(
                hc, W2_ref[0, k0 * HC:(k0 + 1) * HC, :].astype(jnp.bfloat16),
                preferred_element_type=jnp.float32)
        out_ref[...] = acc_ref[...] + b2_ref[0, 0, :][None, :]


def _gmm(xs, W1, b1, W2, b2, blk_expert, nb_used):
    gs = pltpu.PrefetchScalarGridSpec(
        num_scalar_prefetch=2,
        grid=(NB,),
        in_specs=[
            pl.BlockSpec((BM, D),
                         lambda b, be, nu: (jnp.where(b < nu[0], b, 0), 0)),
            pl.BlockSpec((1, D, H), lambda b, be, nu: (be[b], 0, 0)),
            pl.BlockSpec((1, 1, H), lambda b, be, nu: (be[b], 0, 0)),
            pl.BlockSpec((1, H, O), lambda b, be, nu: (be[b], 0, 0)),
            pl.BlockSpec((1, 1, O), lambda b, be, nu: (be[b], 0, 0)),
        ],
        out_specs=pl.BlockSpec(
            (BM, O), lambda b, be, nu: (jnp.where(b < nu[0], b, NB), 0)),
        scratch_shapes=[pltpu.VMEM((BM, O), jnp.float32)],
    )
    return pl.pallas_call(
        _gmm_body, grid_spec=gs,
        out_shape=jax.ShapeDtypeStruct(((NB + 1) * BM, O), jnp.float32),
    )(blk_expert, nb_used, xs, W1, b1.reshape(E, 1, H), W2,
      b2.reshape(E, 1, O))


def _sc_combine(ys, pos0, pos1, w0b, w1b):
    mesh = plsc.VectorSubcoreMesh(core_axis_name="c", subcore_axis_name="s")

    @functools.partial(
        pl.kernel, mesh=mesh,
        out_type=jax.ShapeDtypeStruct((T, O), jnp.float32),
        scratch_types=[
            pltpu.VMEM((CPW,), jnp.int32),
            pltpu.VMEM((CPW,), jnp.int32),
            pltpu.VMEM((CPW, LN), jnp.float32),
            pltpu.VMEM((CPW, LN), jnp.float32),
            pltpu.VMEM((CCH, O), jnp.float32),
            pltpu.VMEM((CCH, O), jnp.float32),
            pltpu.VMEM((CCH, O), jnp.float32),
            pltpu.VMEM((CCH, O), jnp.float32),
            pltpu.VMEM((CCH, O), jnp.float32),
            pltpu.VMEM((CCH, O), jnp.float32),
            pltpu.SemaphoreType.DMA,
            pltpu.SemaphoreType.DMA,
        ],
    )
    def k(ys_hbm, p0_hbm, p1_hbm, w0_hbm, w1_hbm, out_hbm,
          idx0_v, idx1_v, w0_v, w1_v, ra0, ra1, rb0, rb1, oa, ob,
          gsem, wsem):
        wid = lax.axis_index("s") * NC + lax.axis_index("c")
        tb = wid * CPW
        pltpu.sync_copy(p0_hbm.at[pl.ds(tb, CPW)], idx0_v)
        pltpu.sync_copy(p1_hbm.at[pl.ds(tb, CPW)], idx1_v)
        pltpu.sync_copy(w0_hbm.at[pl.ds(tb, CPW)], w0_v)
        pltpu.sync_copy(w1_hbm.at[pl.ds(tb, CPW)], w1_v)
        rabufs = (ra0, ra1)
        rbbufs = (rb0, rb1)
        obufs = (oa, ob)
        nch = CPW // CCH

        def start_gather(c):
            sl = pl.ds(c * CCH, CCH)
            return (pltpu.async_copy(ys_hbm.at[idx0_v.at[sl]],
                                     rabufs[c % 2], gsem),
                    pltpu.async_copy(ys_hbm.at[idx1_v.at[sl]],
                                     rbbufs[c % 2], gsem))

        gathers = [None] * nch
        gathers[0] = start_gather(0)
        writes = []
        for c in range(nch):
            ga, gb = gathers[c]
            ga.wait()
            gb.wait()
            if c + 1 < nch:
                gathers[c + 1] = start_gather(c + 1)
            ra = rabufs[c % 2]
            rb = rbbufs[c % 2]
            out_v = obufs[c % 2]
            cbase = c * CCH
            if c >= 2:
                writes[c - 2].wait()

            def tok_body(i, carry):
                w0 = w0_v[cbase + i, :]
                w1 = w1_v[cbase + i, :]
                for j in range(O // LN):
                    sl = pl.ds(j * LN, LN)
                    out_v[i, sl] = w0 * ra[i, sl] + w1 * rb[i, sl]
                return carry

            lax.fori_loop(0, CCH, tok_body, 0)
            writes.append(pltpu.async_copy(
                out_v, out_hbm.at[pl.ds(tb + c * CCH, CCH)], wsem))
        for wcp in writes[-2:]:
            wcp.wait()

    return k(ys, pos0, pos1, w0b, w1b)


def kernel(x, Wg, bg, W1, b1, W2, b2, num_experts_per_tok):
    del num_experts_per_tok
    x2 = x.reshape(T, D)
    p0, p1, w0b, w1b, be, nbu = _router(x2, Wg, bg)
    pos0 = p0.reshape(T)
    pos1 = p1.reshape(T)
    xs = _sc_dispatch(x2, pos0, pos1)
    ys = _gmm(xs, W1, b1, W2, b2, be.reshape(NB), nbu.reshape(1))
    out = _sc_combine(ys, pos0, pos1, w0b, w1b)
    return out.reshape(1, T, O)
